# Initial kernel scaffold; baseline (speedup 1.0000x reference)
import jax
import jax.numpy as jnp
from jax.experimental import pallas as pl


def kernel(sentence, table):
    raise NotImplementedError("write your pallas kernel here")


# speedup vs baseline: 1.4870x; 1.4870x over previous
import jax
import jax.numpy as jnp
from jax import lax
from jax.experimental import pallas as pl
from jax.experimental.pallas import tpu as pltpu
from jax.experimental.pallas import tpu_sc as plsc

SEQ = 200
BATCH = 4096
DIM = 32
NC, NS = 2, 16
NW = NC * NS
BPW = BATCH // NW


def _sc_body(sent_hbm, table_hbm, out_hbm, idx_v, rows0, rows1, acc, sem0, sem1):
    wid = lax.axis_index("s") * NC + lax.axis_index("c")
    base = wid * BPW

    pltpu.sync_copy(sent_hbm.at[:, pl.ds(base, BPW)], idx_v)

    pltpu.async_copy(table_hbm.at[idx_v.at[0]], rows0, sem0)

    zero = jnp.zeros((16,), jnp.float32)

    def zbody(i, c):
        acc[i, pl.ds(0, 16)] = zero
        acc[i, pl.ds(16, 16)] = zero
        return c
    lax.fori_loop(0, BPW, zbody, 0, unroll=8)

    def accum(buf):
        def abody(i, c):
            acc[i, pl.ds(0, 16)] += buf[i, pl.ds(0, 16)]
            acc[i, pl.ds(16, 16)] += buf[i, pl.ds(16, 16)]
            return c
        lax.fori_loop(0, BPW, abody, 0, unroll=8)

    def step(k, c):
        t = 2 * k
        pltpu.async_copy(table_hbm.at[idx_v.at[t + 1]], rows1, sem1)
        pltpu.make_async_copy(table_hbm.at[idx_v.at[t]], rows0, sem0).wait()
        accum(rows0)

        @pl.when(t + 2 < SEQ)
        def _():
            pltpu.async_copy(table_hbm.at[idx_v.at[t + 2]], rows0, sem0)

        pltpu.make_async_copy(table_hbm.at[idx_v.at[t + 1]], rows1, sem1).wait()
        accum(rows1)
        return c
    lax.fori_loop(0, SEQ // 2, step, 0)

    inv = jnp.float32(1.0 / SEQ)

    def sbody(i, c):
        acc[i, pl.ds(0, 16)] *= inv
        acc[i, pl.ds(16, 16)] *= inv
        return c
    lax.fori_loop(0, BPW, sbody, 0, unroll=8)

    pltpu.sync_copy(acc, out_hbm.at[pl.ds(base, BPW), :])


def kernel(sentence, table):
    k = pl.kernel(
        _sc_body,
        out_type=jax.ShapeDtypeStruct((BATCH, DIM), jnp.float32),
        mesh=plsc.VectorSubcoreMesh(core_axis_name="c", subcore_axis_name="s"),
        compiler_params=pltpu.CompilerParams(use_tc_tiling_on_sc=False),
        scratch_types=[
            pltpu.VMEM((SEQ, BPW), jnp.int32),
            pltpu.VMEM((BPW, DIM), jnp.float32),
            pltpu.VMEM((BPW, DIM), jnp.float32),
            pltpu.VMEM((BPW, DIM), jnp.float32),
            pltpu.SemaphoreType.DMA,
            pltpu.SemaphoreType.DMA,
        ],
    )
    return k(sentence, table)


# speedup vs baseline: 1.9670x; 1.3228x over previous
import jax
import jax.numpy as jnp
from jax import lax
from jax.experimental import pallas as pl
from jax.experimental.pallas import tpu as pltpu
from jax.experimental.pallas import tpu_sc as plsc

SEQ = 200
BATCH = 4096
DIM = 32
NC, NS = 2, 16
NW = NC * NS
BPW = BATCH // NW
NB = 4


def _sc_body(sent_hbm, table_hbm, out_hbm, idx_v, a0, a1, a2, a3,
             s0, s1, s2, s3):
    bufs = (a0, a1, a2, a3)
    sems = (s0, s1, s2, s3)
    wid = lax.axis_index("s") * NC + lax.axis_index("c")
    base = wid * BPW

    pltpu.sync_copy(sent_hbm.at[:, pl.ds(base, BPW)], idx_v)

    for b in range(NB):
        pltpu.async_copy(table_hbm.at[idx_v.at[b]], bufs[b], sems[b])

    def step(k, c):
        t = NB + NB * k
        for b in range(NB):
            pltpu.make_async_copy(table_hbm.at[idx_v.at[0]], bufs[b], sems[b]).wait()
            pltpu.async_copy(table_hbm.at[idx_v.at[t + b]], bufs[b], sems[b],
                             add=True)
        return c
    lax.fori_loop(0, (SEQ - NB) // NB, step, 0)

    for b in range(NB):
        pltpu.make_async_copy(table_hbm.at[idx_v.at[0]], bufs[b], sems[b]).wait()

    inv = jnp.float32(1.0 / SEQ)

    def fbody(i, c):
        for off in (0, 16):
            s01 = a0[i, pl.ds(off, 16)] + a1[i, pl.ds(off, 16)]
            s23 = a2[i, pl.ds(off, 16)] + a3[i, pl.ds(off, 16)]
            a0[i, pl.ds(off, 16)] = (s01 + s23) * inv
        return c
    lax.fori_loop(0, BPW, fbody, 0, unroll=8)

    pltpu.sync_copy(a0, out_hbm.at[pl.ds(base, BPW), :])


def kernel(sentence, table):
    k = pl.kernel(
        _sc_body,
        out_type=jax.ShapeDtypeStruct((BATCH, DIM), jnp.float32),
        mesh=plsc.VectorSubcoreMesh(core_axis_name="c", subcore_axis_name="s"),
        compiler_params=pltpu.CompilerParams(use_tc_tiling_on_sc=False),
        scratch_types=[
            pltpu.VMEM((SEQ, BPW), jnp.int32),
            pltpu.VMEM((BPW, DIM), jnp.float32),
            pltpu.VMEM((BPW, DIM), jnp.float32),
            pltpu.VMEM((BPW, DIM), jnp.float32),
            pltpu.VMEM((BPW, DIM), jnp.float32),
            pltpu.SemaphoreType.DMA,
            pltpu.SemaphoreType.DMA,
            pltpu.SemaphoreType.DMA,
            pltpu.SemaphoreType.DMA,
        ],
    )
    return k(sentence, table)


# speedup vs baseline: 2.0076x; 1.0206x over previous
import jax
import jax.numpy as jnp
from jax import lax
from jax.experimental import pallas as pl
from jax.experimental.pallas import tpu as pltpu
from jax.experimental.pallas import tpu_sc as plsc

SEQ = 200
BATCH = 4096
DIM = 32
NC, NS = 2, 16
NW = NC * NS
BPW = BATCH // NW
NB = 8


def _sc_body(sent_hbm, table_hbm, out_hbm, idx_v, *scr):
    bufs = scr[:NB]
    sems = scr[NB:]
    wid = lax.axis_index("s") * NC + lax.axis_index("c")
    base = wid * BPW

    pltpu.sync_copy(sent_hbm.at[:, pl.ds(base, BPW)], idx_v)

    for b in range(NB):
        pltpu.async_copy(table_hbm.at[idx_v.at[b]], bufs[b], sems[b])

    def step(k, c):
        t = NB + NB * k
        for b in range(NB):
            pltpu.make_async_copy(table_hbm.at[idx_v.at[0]], bufs[b], sems[b]).wait()
            pltpu.async_copy(table_hbm.at[idx_v.at[t + b]], bufs[b], sems[b],
                             add=True)
        return c
    lax.fori_loop(0, (SEQ - NB) // NB, step, 0)

    for b in range(NB):
        pltpu.make_async_copy(table_hbm.at[idx_v.at[0]], bufs[b], sems[b]).wait()

    inv = jnp.float32(1.0 / SEQ)

    def fbody(i, c):
        for off in (0, 16):
            vals = [buf[i, pl.ds(off, 16)] for buf in bufs]
            while len(vals) > 1:
                vals = [vals[j] + vals[j + 1] for j in range(0, len(vals) - 1, 2)] \
                    + ([vals[-1]] if len(vals) % 2 else [])
            bufs[0][i, pl.ds(off, 16)] = vals[0] * inv
        return c
    lax.fori_loop(0, BPW, fbody, 0, unroll=8)

    pltpu.sync_copy(bufs[0], out_hbm.at[pl.ds(base, BPW), :])


def kernel(sentence, table):
    k = pl.kernel(
        _sc_body,
        out_type=jax.ShapeDtypeStruct((BATCH, DIM), jnp.float32),
        mesh=plsc.VectorSubcoreMesh(core_axis_name="c", subcore_axis_name="s"),
        compiler_params=pltpu.CompilerParams(use_tc_tiling_on_sc=False),
        scratch_types=(
            [pltpu.VMEM((SEQ, BPW), jnp.int32)]
            + [pltpu.VMEM((BPW, DIM), jnp.float32)] * NB
            + [pltpu.SemaphoreType.DMA] * NB
        ),
    )
    return k(sentence, table)
